# Initial kernel scaffold; baseline (speedup 1.0000x reference)
import jax
import jax.numpy as jnp
from jax.experimental import pallas as pl


def kernel(boxes, scores, classes, W1, b1, W2, b2, W3, b3, LW1, Lb1, LW2, Lb2):
    raise NotImplementedError("write your pallas kernel here")


# speedup vs baseline: 1.0826x; 1.0826x over previous
import functools

import jax
import jax.numpy as jnp
from jax.experimental import pallas as pl
from jax.experimental.pallas import tpu as pltpu

N_KEEP = 1000
NPAD = 1024
H1 = 32
H2 = 16
LH = 16


def _suppress_kernel(ti, tj, nj,
                     bcol, brow, scol, srow, ccol, crow,
                     w1, b1, w2, b2, w3, b3, lw1, lb1, lw2, lb2,
                     out, accS, accD):
    j = pl.program_id(1)

    xi = bcol[:, 0:1]
    yi = bcol[:, 1:2]
    wi = bcol[:, 2:3]
    hi = bcol[:, 3:4]
    xj = brow[0:1, :]
    yj = brow[1:2, :]
    wj = brow[2:3, :]
    hj = brow[3:4, :]
    si = scol[:, :]
    sj = srow[:, :]

    ai = wi * hi
    aj = wj * hj
    ix = jnp.minimum(xi + wi, xj + wj) - jnp.maximum(xi, xj)
    iy = jnp.minimum(yi + hi, yj + hj) - jnp.maximum(yi, yj)
    inter = jnp.maximum(ix, 0.0) * jnp.maximum(iy, 0.0)
    iou = inter / ((ai + aj - inter) + 1e-6)

    dx = jnp.abs(xi - xj)
    dy = jnp.abs(yi - yj)
    dw = jnp.abs(wi - wj)
    dh = jnp.abs(hi - hj)

    acc = [None] * H2
    for u in range(H1):
        rowv = sj * w1[6, u]
        colv = si * w1[5, u] + b1[0, u]
        a = iou * w1[0, u] + dx * w1[1, u] + dy * w1[2, u] \
            + dw * w1[3, u] + dh * w1[4, u]
        h = jnp.maximum(a + (rowv + colv), 0.0)
        for v in range(H2):
            t = h * w2[u, v]
            acc[v] = t if acc[v] is None else acc[v] + t
    a3 = jnp.full((ti, tj), b3[0, 0], dtype=jnp.float32)
    for v in range(H2):
        a3 = a3 + jnp.maximum(acc[v] + b2[0, v], 0.0) * w3[v, 0]
    s = 1.0 / (1.0 + jnp.exp(-a3))
    s = jnp.where(ccol[:, :] == crow[:, :], s, 0.0)

    @pl.when(j == 0)
    def _init():
        accS[:, :] = jnp.zeros_like(accS)
        accD[:, :] = jnp.zeros_like(accD)

    accS[:, :] += jnp.sum(s * iou, axis=1, keepdims=True)
    accD[:, :] += jnp.sum(iou, axis=1, keepdims=True)

    @pl.when(j == nj - 1)
    def _finalize():
        S = accS[:, :]
        D = accD[:, :] * (1.0 / N_KEEP)
        lacc = None
        for v in range(LH):
            lv = jnp.maximum(
                xi * lw1[0, v] + yi * lw1[1, v] + wi * lw1[2, v]
                + hi * lw1[3, v] + si * lw1[4, v] + lb1[0, v], 0.0)
            t = lv * lw2[v, 0]
            lacc = t if lacc is None else lacc + t
        lam = 1.0 / (1.0 + jnp.exp(-(lacc + lb2[0, 0])))
        out[:, :] = si * jnp.exp(-lam * S * D)


@functools.partial(jax.jit, static_argnames=("ti", "tj"))
def _suppress(bp, sp, ccol, crow, W1, b1, W2, b2, W3, b3, LW1, Lb1, LW2, Lb2,
              ti=8, tj=512):
    ni = NPAD // ti
    nj = NPAD // tj
    bT = bp.T
    scol = sp[:, None]
    srow = sp[None, :]
    smem = functools.partial(pl.BlockSpec, memory_space=pltpu.SMEM)
    grid = (ni, nj)
    in_specs = [
            pl.BlockSpec((ti, 4), lambda i, j: (i, 0)),
            pl.BlockSpec((4, tj), lambda i, j: (0, j)),
            pl.BlockSpec((ti, 1), lambda i, j: (i, 0)),
            pl.BlockSpec((1, tj), lambda i, j: (0, j)),
            pl.BlockSpec((ti, 1), lambda i, j: (i, 0)),
            pl.BlockSpec((1, tj), lambda i, j: (0, j)),
            smem(), smem(), smem(), smem(), smem(), smem(),
            smem(), smem(), smem(), smem(),
        ]
    return pl.pallas_call(
        functools.partial(_suppress_kernel, ti, tj, nj),
        grid=grid,
        in_specs=in_specs,
        out_specs=pl.BlockSpec((ti, 1), lambda i, j: (i, 0)),
        out_shape=jax.ShapeDtypeStruct((NPAD, 1), jnp.float32),
        scratch_shapes=[
            pltpu.VMEM((ti, 1), jnp.float32),
            pltpu.VMEM((ti, 1), jnp.float32),
        ],
    )(bp, bT, scol, srow, ccol, crow,
      W1, b1[None, :], W2, b2[None, :], W3, b3[None, :],
      LW1, Lb1[None, :], LW2, Lb2[None, :])


def kernel(boxes, scores, classes, W1, b1, W2, b2, W3, b3, LW1, Lb1, LW2, Lb2):
    top_scores, idx = jax.lax.top_k(scores, N_KEEP)
    b = boxes[idx]
    c = classes[idx]

    pad = NPAD - N_KEEP
    bp = jnp.concatenate(
        [b, jnp.broadcast_to(jnp.array([[4.0, 4.0, 0.0, 0.0]], jnp.float32),
                             (pad, 4))], axis=0)
    sp = jnp.concatenate([top_scores, jnp.zeros((pad,), jnp.float32)])
    cf = c.astype(jnp.float32)
    ccol = jnp.concatenate([cf, jnp.full((pad,), -1.0, jnp.float32)])[:, None]
    crow = jnp.concatenate([cf, jnp.full((pad,), -2.0, jnp.float32)])[None, :]

    out = _suppress(bp, sp, ccol, crow,
                    W1, b1, W2, b2, W3, b3, LW1, Lb1, LW2, Lb2)
    return (b, out[:N_KEEP, 0], c)


# speedup vs baseline: 1.9859x; 1.8343x over previous
import functools

import jax
import jax.numpy as jnp
from jax.experimental import pallas as pl
from jax.experimental.pallas import tpu as pltpu

N_KEEP = 1000
NPAD = 1024
H1 = 32
H2 = 16
TI = 8


def _suppress_kernel(tj, nj,
                     bcol, brow, scol, srow, ccol, crow,
                     w1, b1, m2k, b2k, m3k, b3, lw1, lb1, lw2, lb2,
                     out):
    xi = bcol[:, 0:1]
    yi = bcol[:, 1:2]
    wi = bcol[:, 2:3]
    hi = bcol[:, 3:4]
    si = scol[:, :]
    ci = ccol[:, :]
    ai = wi * hi
    xiw = xi + wi
    yih = yi + hi

    m2 = m2k[:, :]
    m3 = m3k[:, :]
    b2c = b2k[:, :]

    ious = []
    Hs = []
    for c in range(nj):
        sl = pl.ds(c * tj, tj)
        xj = brow[0:1, sl]
        yj = brow[1:2, sl]
        wj = brow[2:3, sl]
        hj = brow[3:4, sl]
        sj = srow[0:1, sl]

        aj = wj * hj
        ix = jnp.minimum(xiw, xj + wj) - jnp.maximum(xi, xj)
        iy = jnp.minimum(yih, yj + hj) - jnp.maximum(yi, yj)
        inter = jnp.maximum(ix, 0.0) * jnp.maximum(iy, 0.0)
        iou = inter / ((ai + aj - inter) + 1e-6)
        ious.append(iou)

        dx = jnp.abs(xi - xj)
        dy = jnp.abs(yi - yj)
        dw = jnp.abs(wi - wj)
        dh = jnp.abs(hi - hj)

        hs = []
        for u in range(H1):
            rowv = sj * w1[6, u]
            colv = si * w1[5, u] + b1[0, u]
            a = iou * w1[0, u] + dx * w1[1, u] + dy * w1[2, u] \
                + dw * w1[3, u] + dh * w1[4, u]
            hs.append(jnp.maximum(a + (rowv + colv), 0.0)
                      .astype(jnp.bfloat16))
        Hs.append(jnp.concatenate(hs, axis=0))

    H = jnp.concatenate(Hs, axis=1)
    iou_full = jnp.concatenate(ious, axis=1)
    h2 = jnp.dot(m2, H, preferred_element_type=jnp.float32)
    h2 = jnp.maximum(h2 + b2c, 0.0).astype(jnp.bfloat16)
    a3 = jnp.dot(m3, h2, preferred_element_type=jnp.float32) + b3[0, 0]
    s = 1.0 / (1.0 + jnp.exp(-a3))
    s = jnp.where(ci == crow[:, :], s, 0.0)

    p = s * iou_full
    S = jnp.sum(p, axis=1, keepdims=True)
    D = jnp.sum(iou_full, axis=1, keepdims=True) * (1.0 / N_KEEP)

    lam_in = jnp.concatenate([bcol[:, :], si], axis=1)
    l1 = jnp.maximum(
        jnp.dot(lam_in, lw1[:, :], preferred_element_type=jnp.float32)
        + lb1[:, :], 0.0)
    lam_pre = jnp.dot(l1, lw2[:, :], preferred_element_type=jnp.float32) \
        + lb2[:, :]
    lam = 1.0 / (1.0 + jnp.exp(-lam_pre))
    out[:, :] = si * jnp.exp(-lam * S * D)


@functools.partial(jax.jit, static_argnames=("tj",))
def _suppress(bp, sp, ccol, crow, W1, b1, W2, b2, W3, b3, LW1, Lb1, LW2, Lb2,
              tj=256):
    ni = NPAD // TI
    nj = NPAD // tj
    bT = bp.T
    scol = sp[:, None]
    srow = sp[None, :]
    smem = functools.partial(pl.BlockSpec, memory_space=pltpu.SMEM)
    whole = lambda shape: pl.BlockSpec(shape, lambda i: tuple(0 for _ in shape))
    grid = (ni,)
    in_specs = [
        pl.BlockSpec((TI, 4), lambda i: (i, 0)),
        whole((4, NPAD)),
        pl.BlockSpec((TI, 1), lambda i: (i, 0)),
        whole((1, NPAD)),
        pl.BlockSpec((TI, 1), lambda i: (i, 0)),
        whole((1, NPAD)),
        smem(), smem(),
        whole((TI * H2, TI * H1)),
        whole((TI * H2, 1)),
        whole((TI, TI * H2)),
        smem(),
        whole((5, 16)), whole((1, 16)),
        whole((16, 1)), whole((1, 1)),
    ]
    eye = jnp.eye(TI, dtype=jnp.float32)
    m2k = jnp.kron(W2.T, eye).astype(jnp.bfloat16)
    b2k = jnp.repeat(b2, TI)[:, None]
    m3k = jnp.kron(W3.T, eye).astype(jnp.bfloat16)
    return pl.pallas_call(
        functools.partial(_suppress_kernel, tj, nj),
        grid=grid,
        in_specs=in_specs,
        out_specs=pl.BlockSpec((TI, 1), lambda i: (i, 0)),
        out_shape=jax.ShapeDtypeStruct((NPAD, 1), jnp.float32),
    )(bp, bT, scol, srow, ccol, crow,
      W1, b1[None, :], m2k, b2k, m3k, b3[None, :],
      LW1, Lb1[None, :], LW2, Lb2[None, :])


def kernel(boxes, scores, classes, W1, b1, W2, b2, W3, b3, LW1, Lb1, LW2, Lb2):
    top_scores, idx = jax.lax.top_k(scores, N_KEEP)
    b = boxes[idx]
    c = classes[idx]

    pad = NPAD - N_KEEP
    bp = jnp.concatenate(
        [b, jnp.broadcast_to(jnp.array([[4.0, 4.0, 0.0, 0.0]], jnp.float32),
                             (pad, 4))], axis=0)
    sp = jnp.concatenate([top_scores, jnp.zeros((pad,), jnp.float32)])
    cf = c.astype(jnp.float32)
    ccol = jnp.concatenate([cf, jnp.full((pad,), -1.0, jnp.float32)])[:, None]
    crow = jnp.concatenate([cf, jnp.full((pad,), -2.0, jnp.float32)])[None, :]

    out = _suppress(bp, sp, ccol, crow,
                    W1, b1, W2, b2, W3, b3, LW1, Lb1, LW2, Lb2)
    return (b, out[:N_KEEP, 0], c)
